# baseline (speedup 1.0000x reference)
import functools

import jax
import jax.numpy as jnp
from jax import lax
from jax.experimental import pallas as pl
from jax.experimental.pallas import tpu as pltpu
from jax.experimental.pallas import tpu_sc as plsc

NC, NS, L = 2, 16, 16
NW = NC * NS
CHUNK = 128


@functools.partial(jax.jit, static_argnums=(8, 9))
def _glove_sc(cw, tw, co, wt, emb_v, emb_u, vb, ub, B, D):
    n_per_w = B // NW
    n_chunks = n_per_w // CHUNK
    n_groups = n_per_w // L

    mesh = plsc.VectorSubcoreMesh(core_axis_name="c", subcore_axis_name="s")

    @functools.partial(
        pl.kernel,
        out_type=jax.ShapeDtypeStruct((NW, L), jnp.float32),
        mesh=mesh,
        compiler_params=pltpu.CompilerParams(needs_layout_passes=False,
                                             use_tc_tiling_on_sc=False),
        scratch_types=[
            pltpu.VMEM((n_chunks, CHUNK), jnp.int32),
            pltpu.VMEM((n_chunks, CHUNK), jnp.int32),
            pltpu.VMEM((n_per_w, D), jnp.float32),
            pltpu.VMEM((n_per_w, D), jnp.float32),
            pltpu.VMEM((n_per_w,), jnp.float32),
            pltpu.VMEM((n_per_w,), jnp.float32),
            pltpu.VMEM((n_per_w,), jnp.float32),
            pltpu.VMEM((n_per_w,), jnp.float32),
            pltpu.VMEM((L,), jnp.float32),
            pltpu.SemaphoreType.DMA,
        ],
    )
    def glove_kernel(cw_hbm, tw_hbm, co_hbm, wt_hbm, ev_hbm, eu_hbm,
                     vb_hbm, ub_hbm, out_hbm,
                     idxc, idxt, rowsc, rowst, cb, tb, cov, wv,
                     obuf, sem):
        wid = lax.axis_index("c") * NS + lax.axis_index("s")
        base = wid * n_per_w
        crow = wid * n_chunks

        pltpu.sync_copy(cw_hbm.at[pl.ds(crow, n_chunks)], idxc)
        pltpu.sync_copy(tw_hbm.at[pl.ds(crow, n_chunks)], idxt)
        pltpu.sync_copy(co_hbm.at[pl.ds(base, n_per_w)], cov)
        pltpu.sync_copy(wt_hbm.at[pl.ds(base, n_per_w)], wv)

        copies = []
        for j in range(n_chunks):
            dst = pl.ds(j * CHUNK, CHUNK)
            copies.append(pltpu.async_copy(ev_hbm.at[idxc.at[j]],
                                           rowsc.at[dst], sem))
            copies.append(pltpu.async_copy(eu_hbm.at[idxt.at[j]],
                                           rowst.at[dst], sem))
            copies.append(pltpu.async_copy(vb_hbm.at[idxc.at[j]],
                                           cb.at[dst], sem))
            copies.append(pltpu.async_copy(ub_hbm.at[idxt.at[j]],
                                           tb.at[dst], sem))
        for c in copies:
            c.wait()

        lane = lax.iota(jnp.int32, L)

        def group(g, acc):
            ips = jnp.zeros((L,), jnp.float32)
            for i in range(L):
                p = g * L + i
                s = rowsc[p, pl.ds(0, L)] * rowst[p, pl.ds(0, L)]
                for k in range(1, D // L):
                    s = s + (rowsc[p, pl.ds(k * L, L)]
                             * rowst[p, pl.ds(k * L, L)])
                ips = jnp.where(lane == i, jnp.sum(s), ips)
            gsl = pl.ds(g * L, L)
            err = ips + cb[gsl] + tb[gsl] - cov[gsl]
            return acc + wv[gsl] * err * err

        acc = lax.fori_loop(0, n_groups, group, jnp.zeros((L,), jnp.float32))
        obuf[...] = acc
        pltpu.sync_copy(obuf, out_hbm.at[wid])

    return glove_kernel(cw, tw, co, wt, emb_v, emb_u, vb, ub)


def kernel(center_words, target_words, coocs, weighting,
           emb_v, emb_u, v_bias, u_bias):
    B = center_words.shape[0]
    D = emb_v.shape[1]
    cw = center_words.reshape(B // CHUNK, CHUNK).astype(jnp.int32)
    tw = target_words.reshape(B // CHUNK, CHUNK).astype(jnp.int32)
    partials = _glove_sc(cw, tw, coocs.reshape(-1), weighting.reshape(-1),
                         emb_v, emb_u, v_bias.reshape(-1), u_bias.reshape(-1),
                         B, D)
    return jnp.sum(partials)
